# Initial kernel scaffold; baseline (speedup 1.0000x reference)
import jax
import jax.numpy as jnp
from jax.experimental import pallas as pl


def kernel(worker, worker_COUNT, worker_cargo_full, road_level, resources, input_mask, emb_worker, emb_cargo, conv_w, conv_b):
    raise NotImplementedError("write your pallas kernel here")


# speedup vs baseline: 13.3067x; 13.3067x over previous
import jax
import jax.numpy as jnp
from jax.experimental import pallas as pl

_B, _H, _W = 128, 32, 32
_S = _H * _W
_EMB = 128
_TB = 4


def _body(wk_ref, cnt_ref, cargo_ref, road_ref, res_ref, mask_ref, mt_ref,
          out_ref):
    mt = mt_ref[...]
    for tb in range(_TB):
        mask = mask_ref[tb]
        wk = wk_ref[tb].astype(jnp.float32)
        cnt = cnt_ref[tb]
        res = res_ref[tb]
        f0 = road_ref[tb] * mask
        f1 = res[0:1] * mask
        f2 = res[1:2] * mask
        f3 = res[2:3] * mask
        f4 = wk[0:1] * cnt[0:1] * mask
        f5 = wk[1:2] * cnt[1:2] * mask
        f6 = cargo_ref[tb].astype(jnp.float32) * mask
        acc = jnp.broadcast_to(mt[:, 7:8], (_EMB, _S))
        for i, f in enumerate((f0, f1, f2, f3, f4, f5, f6)):
            acc = acc + mt[:, i:i + 1] * f
        out_ref[tb] = acc


def kernel(worker, worker_COUNT, worker_cargo_full, road_level, resources,
           input_mask, emb_worker, emb_cargo, conv_w, conv_b):
    wk = worker.reshape(_B, 2, _S)
    cnt = worker_COUNT.reshape(_B, 2, _S)
    cargo = worker_cargo_full.reshape(_B, 1, _S)
    road = road_level.reshape(_B, 1, _S)
    res = resources.reshape(_B, 3, _S)
    mask = input_mask.reshape(_B, 1, _S)

    ew1 = emb_worker[1]
    ec1 = emb_cargo[1]
    zeros = jnp.zeros((_EMB // 2,), jnp.float32)
    col4 = jnp.concatenate([ew1, zeros])
    col5 = jnp.concatenate([zeros, ew1])
    mt = jnp.concatenate(
        [conv_w, col4[:, None], col5[:, None], ec1[:, None], conv_b[:, None]],
        axis=1)

    grid = (_B // _TB,)
    bs = lambda k: pl.BlockSpec((_TB, k, _S), lambda i: (i, 0, 0))
    out = pl.pallas_call(
        _body,
        grid=grid,
        in_specs=[
            bs(2), bs(2), bs(1), bs(1), bs(3), bs(1),
            pl.BlockSpec((_EMB, 8), lambda i: (0, 0)),
        ],
        out_specs=pl.BlockSpec((_TB, _EMB, _S), lambda i: (i, 0, 0)),
        out_shape=jax.ShapeDtypeStruct((_B, _EMB, _S), jnp.float32),
    )(wk, cnt, cargo, road, res, mask, mt)
    return out.reshape(_B, _EMB, _H, _W), input_mask


# speedup vs baseline: 13.3211x; 1.0011x over previous
import jax
import jax.numpy as jnp
from jax.experimental import pallas as pl
from jax.experimental.pallas import tpu as pltpu

_B, _H, _W = 128, 32, 32
_S = _H * _W
_EMB = 128
_TB = 4


def _body(wk_ref, cnt_ref, cargo_ref, road_ref, res_ref, mask_ref, mt_ref,
          out_ref):
    mt = mt_ref[...]
    for tb in range(_TB):
        mask = mask_ref[tb]
        wk = wk_ref[tb].astype(jnp.float32)
        cnt = cnt_ref[tb]
        res = res_ref[tb]
        f0 = road_ref[tb] * mask
        f1 = res[0:1] * mask
        f2 = res[1:2] * mask
        f3 = res[2:3] * mask
        f4 = wk[0:1] * cnt[0:1] * mask
        f5 = wk[1:2] * cnt[1:2] * mask
        f6 = cargo_ref[tb].astype(jnp.float32) * mask
        acc = jnp.broadcast_to(mt[:, 7:8], (_EMB, _S))
        for i, f in enumerate((f0, f1, f2, f3, f4, f5, f6)):
            acc = acc + mt[:, i:i + 1] * f
        out_ref[tb] = acc


def kernel(worker, worker_COUNT, worker_cargo_full, road_level, resources,
           input_mask, emb_worker, emb_cargo, conv_w, conv_b):
    wk = worker.reshape(_B, 2, _S)
    cnt = worker_COUNT.reshape(_B, 2, _S)
    cargo = worker_cargo_full.reshape(_B, 1, _S)
    road = road_level.reshape(_B, 1, _S)
    res = resources.reshape(_B, 3, _S)
    mask = input_mask.reshape(_B, 1, _S)

    ew1 = emb_worker[1]
    ec1 = emb_cargo[1]
    zeros = jnp.zeros((_EMB // 2,), jnp.float32)
    col4 = jnp.concatenate([ew1, zeros])
    col5 = jnp.concatenate([zeros, ew1])
    mt = jnp.concatenate(
        [conv_w, col4[:, None], col5[:, None], ec1[:, None], conv_b[:, None]],
        axis=1)

    grid = (_B // _TB,)
    bs = lambda k: pl.BlockSpec((_TB, k, _S), lambda i: (i, 0, 0))
    out = pl.pallas_call(
        _body,
        grid=grid,
        in_specs=[
            bs(2), bs(2), bs(1), bs(1), bs(3), bs(1),
            pl.BlockSpec((_EMB, 8), lambda i: (0, 0)),
        ],
        out_specs=pl.BlockSpec((_TB, _EMB, _S), lambda i: (i, 0, 0)),
        out_shape=jax.ShapeDtypeStruct((_B, _EMB, _S), jnp.float32),
        compiler_params=pltpu.CompilerParams(
            dimension_semantics=("parallel",)),
    )(wk, cnt, cargo, road, res, mask, mt)
    return out.reshape(_B, _EMB, _H, _W), input_mask
